# baseline (speedup 1.0000x reference)
import functools

import jax
import jax.numpy as jnp
from jax import lax
from jax.experimental import pallas as pl
from jax.experimental.pallas import tpu as pltpu
from jax.experimental.pallas import tpu_sc as plsc

K = 2
TM = 512
NH = 4
SC_CH = 96


def _sc_row_gather(table, idx):
    _, d = table.shape
    b = idx.shape[0]
    info = plsc.get_sparse_core_info()
    nw = info.num_cores * info.num_subcores
    bpw = b // nw
    pieces = []
    off = 0
    while off < bpw:
        sz = min(SC_CH, bpw - off)
        pieces.append((off, sz))
        off += sz
    mesh = plsc.VectorSubcoreMesh(core_axis_name="c", subcore_axis_name="s")

    @functools.partial(
        pl.kernel, mesh=mesh,
        out_type=jax.ShapeDtypeStruct((b, d), table.dtype),
        scratch_types=[
            pltpu.VMEM((bpw,), jnp.int32),
            pltpu.VMEM((SC_CH, d), table.dtype),
            pltpu.SemaphoreType.DMA,
        ],
    )
    def k(table_hbm, idx_hbm, out_hbm, idx_v, rows_v, sem):
        wid = lax.axis_index("s") * info.num_cores + lax.axis_index("c")
        base = wid * bpw
        pltpu.sync_copy(idx_hbm.at[pl.ds(base, bpw)], idx_v)
        for off, sz in pieces:
            pltpu.async_copy(table_hbm.at[idx_v.at[pl.ds(off, sz)]],
                             rows_v.at[pl.ds(0, sz)], sem).wait()
            pltpu.sync_copy(rows_v.at[pl.ds(0, sz)],
                            out_hbm.at[pl.ds(base + off, sz)])

    return k(table, idx)


def _gemm_body(eid_ref, rs_ref, nv_ref, xs_ref, p_ref,
               w1_ref, w3_ref, w2_ref, ys_ref, hh_ref, *, nh, hb):
    w = pl.program_id(0)
    h = pl.program_id(1)
    nv = nv_ref[w]
    rs = pl.multiple_of(rs_ref[w], 8)

    @pl.when(nv > 0)
    def _():
        @pl.when(h < nh)
        def _():
            xb = xs_ref[pl.ds(rs, TM), :]
            w1b = w1_ref[0].astype(jnp.bfloat16)
            w3b = w3_ref[0].astype(jnp.bfloat16)
            g = jnp.dot(xb, w1b, preferred_element_type=jnp.float32)
            u = jnp.dot(xb, w3b, preferred_element_type=jnp.float32)
            hh_ref[h] = (g * jax.nn.sigmoid(g) * u).astype(jnp.bfloat16)

        @pl.when(h == nh)
        def _():
            part = jnp.zeros((TM, w2_ref.shape[2]), jnp.float32)
            for j in range(nh):
                w2b = w2_ref[0, pl.ds(j * hb, hb), :].astype(jnp.bfloat16)
                part = part + jnp.dot(hh_ref[j], w2b,
                                      preferred_element_type=jnp.float32)
            p_blk = p_ref[pl.ds(rs, TM), :]
            ys_ref[pl.ds(rs, TM), :] = (part * p_blk).astype(jnp.bfloat16)


def _pair_add_body(za_ref, zb_ref, out_ref):
    out_ref[...] = za_ref[...].astype(jnp.float32) + zb_ref[...].astype(jnp.float32)


def kernel(x, gate_w, w1, w3, w2):
    b, s, d = x.shape
    e_num, _, hid = w1.shape
    t_num = b * s
    r_num = t_num * K
    r_pad = ((r_num + 8 * (e_num - 1) + TM - 8 + 255) // 256) * 256
    xf = x.reshape(t_num, d)

    scores = xf @ gate_w.T
    expert_weights, expert_indices = jax.lax.top_k(scores, K)
    expert_weights = jax.nn.softmax(expert_weights, axis=-1)

    ef = expert_indices.reshape(-1).astype(jnp.int32)
    onehot = (ef[:, None] == jnp.arange(e_num, dtype=jnp.int32)[None, :])
    cum = jnp.cumsum(onehot.astype(jnp.int32), axis=0)
    counts = cum[-1].astype(jnp.int32)
    pc = ((counts + 7) // 8) * 8
    pstart = (jnp.cumsum(pc) - pc).astype(jnp.int32)
    rank = jnp.take_along_axis(cum, ef[:, None], axis=1)[:, 0] - 1
    slot = (pstart[ef] + rank).astype(jnp.int32)
    tokf = jnp.arange(r_num, dtype=jnp.int32) // K
    tok2 = jnp.zeros((r_pad,), jnp.int32).at[slot].set(tokf)
    p2 = jnp.zeros((r_pad,), jnp.float32).at[slot].set(
        expert_weights.reshape(-1))[:, None]
    inv3 = jnp.concatenate([slot[0::2], slot[1::2]])

    maxj = r_num // TM
    w_items = e_num + maxj - 1
    nch = (pc + TM - 1) // TM
    e_c = jnp.repeat(jnp.arange(e_num, dtype=jnp.int32), maxj)
    j_c = jnp.tile(jnp.arange(maxj, dtype=jnp.int32), e_num)
    validc = j_c < nch[e_c]
    ordc = jnp.argsort(jnp.logical_not(validc).astype(jnp.int32))[:w_items]
    v_w = validc[ordc]
    eidw = jnp.where(v_w, e_c[ordc], e_num - 1).astype(jnp.int32)
    rsw = jnp.where(v_w, pstart[e_c[ordc]] + j_c[ordc] * TM, 0).astype(jnp.int32)
    nvw = (jnp.clip((pstart + counts)[e_c[ordc]] - rsw, 0, TM)
           * v_w).astype(jnp.int32)

    xf_i = jax.lax.bitcast_convert_type(
        xf.astype(jnp.bfloat16).reshape(t_num, d // 2, 2), jnp.int32)
    xs = jax.lax.bitcast_convert_type(
        _sc_row_gather(xf_i, tok2), jnp.bfloat16).reshape(r_pad, d)

    hb = hid // NH
    grid_spec = pltpu.PrefetchScalarGridSpec(
        num_scalar_prefetch=3,
        grid=(w_items, NH + 1),
        in_specs=[
            pl.BlockSpec((r_pad, d), lambda w, h, eid, rs, nv: (0, 0)),
            pl.BlockSpec((r_pad, 1), lambda w, h, eid, rs, nv: (0, 0)),
            pl.BlockSpec((1, d, hb),
                         lambda w, h, eid, rs, nv: (eid[w], 0, jnp.minimum(h, NH - 1))),
            pl.BlockSpec((1, d, hb),
                         lambda w, h, eid, rs, nv: (eid[w], 0, jnp.minimum(h, NH - 1))),
            pl.BlockSpec((1, hid, d), lambda w, h, eid, rs, nv: (eid[w], 0, 0)),
        ],
        out_specs=pl.BlockSpec((r_pad, d), lambda w, h, eid, rs, nv: (0, 0)),
        scratch_shapes=[pltpu.VMEM((NH, TM, hb), jnp.bfloat16)],
    )
    ys = pl.pallas_call(
        functools.partial(_gemm_body, nh=NH, hb=hb),
        grid_spec=grid_spec,
        out_shape=jax.ShapeDtypeStruct((r_pad, d), jnp.bfloat16),
        compiler_params=pltpu.CompilerParams(
            dimension_semantics=("arbitrary", "arbitrary"),
            vmem_limit_bytes=100 * 1024 * 1024,
        ),
    )(eidw, rsw, nvw, xs, p2, w1, w3, w2)

    ys_i = jax.lax.bitcast_convert_type(
        ys.reshape(r_pad, d // 2, 2), jnp.int32)
    z = jax.lax.bitcast_convert_type(
        _sc_row_gather(ys_i, inv3), jnp.bfloat16).reshape(r_num, d)

    nt = t_num // TM
    out = pl.pallas_call(
        _pair_add_body,
        grid=(nt,),
        in_specs=[
            pl.BlockSpec((TM, d), lambda i: (i, 0)),
            pl.BlockSpec((TM, d), lambda i: (i + nt, 0)),
        ],
        out_specs=pl.BlockSpec((TM, d), lambda i: (i, 0)),
        out_shape=jax.ShapeDtypeStruct((t_num, d), jnp.float32),
    )(z, z)

    return out.reshape(b, s, d)


# speedup vs baseline: 1.6446x; 1.6446x over previous
import functools

import jax
import jax.numpy as jnp
from jax import lax
from jax.experimental import pallas as pl
from jax.experimental.pallas import tpu as pltpu
from jax.experimental.pallas import tpu_sc as plsc

K = 2
TM = 256
NH = 4
SC_CH = 96


def _sc_row_gather(table, idx):
    _, d = table.shape
    b = idx.shape[0]
    info = plsc.get_sparse_core_info()
    nw = info.num_cores * info.num_subcores
    bpw = b // nw
    pieces = []
    off = 0
    while off < bpw:
        sz = min(SC_CH, bpw - off)
        pieces.append((off, sz))
        off += sz
    mesh = plsc.VectorSubcoreMesh(core_axis_name="c", subcore_axis_name="s")

    @functools.partial(
        pl.kernel, mesh=mesh,
        out_type=jax.ShapeDtypeStruct((b, d), table.dtype),
        scratch_types=[
            pltpu.VMEM((bpw,), jnp.int32),
            pltpu.VMEM((SC_CH, d), table.dtype),
            pltpu.SemaphoreType.DMA,
        ],
    )
    def k(table_hbm, idx_hbm, out_hbm, idx_v, rows_v, sem):
        wid = lax.axis_index("s") * info.num_cores + lax.axis_index("c")
        base = wid * bpw
        pltpu.sync_copy(idx_hbm.at[pl.ds(base, bpw)], idx_v)
        for off, sz in pieces:
            pltpu.async_copy(table_hbm.at[idx_v.at[pl.ds(off, sz)]],
                             rows_v.at[pl.ds(0, sz)], sem).wait()
            pltpu.sync_copy(rows_v.at[pl.ds(0, sz)],
                            out_hbm.at[pl.ds(base + off, sz)])

    return k(table, idx)


def _gemm_body(eid_ref, rsb_ref, nv_ref, xs_ref, p_ref,
               w1_ref, w3_ref, w2_ref, ys_ref, hh_ref, *, nh, hb):
    h = pl.program_id(0)
    w = pl.program_id(1)
    nv = nv_ref[w]

    @pl.when(nv > 0)
    def _():
        @pl.when(h < nh)
        def _():
            xb = xs_ref[...].astype(jnp.bfloat16)
            w1b = w1_ref[0].astype(jnp.bfloat16)
            w3b = w3_ref[0].astype(jnp.bfloat16)
            g = jnp.dot(xb, w1b, preferred_element_type=jnp.float32)
            u = jnp.dot(xb, w3b, preferred_element_type=jnp.float32)
            hh_ref[w, h] = (g * jax.nn.sigmoid(g) * u).astype(jnp.bfloat16)

        @pl.when(h == nh)
        def _():
            part = jnp.zeros((TM, w2_ref.shape[2]), jnp.float32)
            for j in range(nh):
                w2b = w2_ref[0, pl.ds(j * hb, hb), :].astype(jnp.bfloat16)
                part = part + jnp.dot(hh_ref[w, j], w2b,
                                      preferred_element_type=jnp.float32)
            ys_ref[...] = part * p_ref[...]


def _pair_add_body(za_ref, zb_ref, out_ref):
    out_ref[...] = za_ref[...] + zb_ref[...]


def kernel(x, gate_w, w1, w3, w2):
    b, s, d = x.shape
    e_num, _, hid = w1.shape
    t_num = b * s
    r_num = t_num * K
    r_pad = r_num + e_num * TM
    n_blk = r_pad // TM
    xf = x.reshape(t_num, d)

    scores = xf @ gate_w.T
    expert_weights, expert_indices = jax.lax.top_k(scores, K)
    expert_weights = jax.nn.softmax(expert_weights, axis=-1)

    ef = expert_indices.reshape(-1).astype(jnp.int32)
    onehot = (ef[:, None] == jnp.arange(e_num, dtype=jnp.int32)[None, :])
    cum = jnp.cumsum(onehot.astype(jnp.int32), axis=0)
    counts = cum[-1].astype(jnp.int32)
    pc = ((counts + TM - 1) // TM) * TM
    pstart = (jnp.cumsum(pc) - pc).astype(jnp.int32)
    rank = jnp.take_along_axis(cum, ef[:, None], axis=1)[:, 0] - 1
    slot = (pstart[ef] + rank).astype(jnp.int32)
    tokf = jnp.arange(r_num, dtype=jnp.int32) // K
    tok2 = jnp.zeros((r_pad,), jnp.int32).at[slot].set(tokf)
    p2 = jnp.zeros((r_pad,), jnp.float32).at[slot].set(
        expert_weights.reshape(-1))[:, None]
    inv3 = jnp.concatenate([slot[0::2], slot[1::2]])

    maxj = r_num // TM
    w_items = e_num + maxj - 1
    nch = pc // TM
    e_c = jnp.repeat(jnp.arange(e_num, dtype=jnp.int32), maxj)
    j_c = jnp.tile(jnp.arange(maxj, dtype=jnp.int32), e_num)
    validc = j_c < nch[e_c]
    ordc = jnp.argsort(jnp.logical_not(validc).astype(jnp.int32))[:w_items]
    v_w = validc[ordc]
    eidw = jnp.where(v_w, e_c[ordc], e_num - 1).astype(jnp.int32)
    rsbw = jnp.where(v_w, pstart[e_c[ordc]] // TM + j_c[ordc], 0).astype(jnp.int32)
    nvw = (jnp.clip(counts[e_c[ordc]] - j_c[ordc] * TM, 0, TM)
           * v_w).astype(jnp.int32)

    xs = _sc_row_gather(xf, tok2)

    hb = hid // NH
    dead = n_blk - 1
    grid_spec = pltpu.PrefetchScalarGridSpec(
        num_scalar_prefetch=3,
        grid=(NH + 1, w_items),
        in_specs=[
            pl.BlockSpec((TM, d), lambda h, w, eid, rsb, nv: (rsb[w], 0)),
            pl.BlockSpec((TM, 1), lambda h, w, eid, rsb, nv: (rsb[w], 0)),
            pl.BlockSpec((1, d, hb),
                         lambda h, w, eid, rsb, nv:
                         (eid[w], 0, jnp.minimum(h, NH - 1))),
            pl.BlockSpec((1, d, hb),
                         lambda h, w, eid, rsb, nv:
                         (eid[w], 0, jnp.minimum(h, NH - 1))),
            pl.BlockSpec((1, hid, d),
                         lambda h, w, eid, rsb, nv:
                         (jnp.where(h == NH, eid[w], 0), 0, 0)),
        ],
        out_specs=pl.BlockSpec(
            (TM, d),
            lambda h, w, eid, rsb, nv:
            (jnp.where(jnp.logical_and(h == NH, nv[w] > 0), rsb[w], dead), 0)),
        scratch_shapes=[pltpu.VMEM((w_items, NH, TM, hb), jnp.bfloat16)],
    )
    ys = pl.pallas_call(
        functools.partial(_gemm_body, nh=NH, hb=hb),
        grid_spec=grid_spec,
        out_shape=jax.ShapeDtypeStruct((r_pad, d), jnp.float32),
        compiler_params=pltpu.CompilerParams(
            dimension_semantics=("arbitrary", "arbitrary"),
            vmem_limit_bytes=100 * 1024 * 1024,
        ),
    )(eidw, rsbw, nvw, xs, p2, w1, w3, w2)

    z = _sc_row_gather(ys, inv3)

    nt = t_num // 512
    out = pl.pallas_call(
        _pair_add_body,
        grid=(nt,),
        in_specs=[
            pl.BlockSpec((512, d), lambda i: (i, 0)),
            pl.BlockSpec((512, d), lambda i: (i + nt, 0)),
        ],
        out_specs=pl.BlockSpec((512, d), lambda i: (i, 0)),
        out_shape=jax.ShapeDtypeStruct((t_num, d), jnp.float32),
    )(z, z)

    return out.reshape(b, s, d)


# speedup vs baseline: 2.2085x; 1.3429x over previous
import functools

import jax
import jax.numpy as jnp
from jax.experimental import pallas as pl
from jax.experimental.pallas import tpu as pltpu

K = 2
TM = 512
NH = 4


def _moe_body(eid_ref, rs_ref, nv_ref, tok_ref, p_ref,
              xf_ref, w1_ref, w3_ref, w2_ref, out_ref,
              xs_ref, hh_ref, acc_ref, *, nh, hb):
    w = pl.program_id(0)
    h = pl.program_id(1)
    nv = nv_ref[w]
    rs = rs_ref[w]

    @pl.when(jnp.logical_and(w == 0, h == 0))
    def _():
        out_ref[...] = jnp.zeros_like(out_ref)

    @pl.when(nv > 0)
    def _():
        @pl.when(h == 0)
        def _():
            def gather_row(i, carry):
                t = tok_ref[rs + i]
                xs_ref[pl.ds(i, 1), :] = xf_ref[pl.ds(t, 1), :]
                return carry
            jax.lax.fori_loop(0, nv, gather_row, 0)

        @pl.when(h < nh)
        def _():
            xb = xs_ref[...].astype(jnp.bfloat16)
            w1b = w1_ref[0].astype(jnp.bfloat16)
            w3b = w3_ref[0].astype(jnp.bfloat16)
            g = jnp.dot(xb, w1b, preferred_element_type=jnp.float32)
            u = jnp.dot(xb, w3b, preferred_element_type=jnp.float32)
            hh_ref[h] = (g * jax.nn.sigmoid(g) * u).astype(jnp.bfloat16)

        @pl.when(h == nh)
        def _():
            part = jnp.zeros((TM, out_ref.shape[1]), jnp.float32)
            for j in range(nh):
                w2b = w2_ref[0, pl.ds(j * hb, hb), :].astype(jnp.bfloat16)
                part = part + jnp.dot(hh_ref[j], w2b,
                                      preferred_element_type=jnp.float32)
            acc_ref[...] = part

            def scatter_row(i, carry):
                r = rs + i
                t = tok_ref[r]
                out_ref[pl.ds(t, 1), :] += p_ref[r] * acc_ref[pl.ds(i, 1), :]
                return carry
            jax.lax.fori_loop(0, nv, scatter_row, 0)


def kernel(x, gate_w, w1, w3, w2):
    b, s, d = x.shape
    e_num, _, hid = w1.shape
    t_num = b * s
    r_num = t_num * K
    xf = x.reshape(t_num, d)

    scores = xf @ gate_w.T
    expert_weights, expert_indices = jax.lax.top_k(scores, K)
    expert_weights = jax.nn.softmax(expert_weights, axis=-1)

    ef = expert_indices.reshape(-1).astype(jnp.int32)
    onehot = (ef[:, None] == jnp.arange(e_num, dtype=jnp.int32)[None, :])
    cum = jnp.cumsum(onehot.astype(jnp.int32), axis=0)
    counts = cum[-1].astype(jnp.int32)
    starts = (jnp.cumsum(counts) - counts).astype(jnp.int32)
    rank = jnp.take_along_axis(cum, ef[:, None], axis=1)[:, 0] - 1
    slot = (starts[ef] + rank).astype(jnp.int32)
    tokf = jnp.arange(r_num, dtype=jnp.int32) // K
    tok_s = jnp.zeros((r_num,), jnp.int32).at[slot].set(tokf)
    p_s = jnp.zeros((r_num,), jnp.float32).at[slot].set(
        expert_weights.reshape(-1))

    maxj = r_num // TM
    w_items = e_num + maxj - 1
    nch = (counts + TM - 1) // TM
    e_c = jnp.repeat(jnp.arange(e_num, dtype=jnp.int32), maxj)
    j_c = jnp.tile(jnp.arange(maxj, dtype=jnp.int32), e_num)
    validc = j_c < nch[e_c]
    ordc = jnp.argsort(jnp.logical_not(validc).astype(jnp.int32))[:w_items]
    v_w = validc[ordc]
    eidw = jnp.where(v_w, e_c[ordc], e_num - 1).astype(jnp.int32)
    rsw = jnp.where(v_w, starts[e_c[ordc]] + j_c[ordc] * TM, 0).astype(jnp.int32)
    nvw = (jnp.clip(counts[e_c[ordc]] - j_c[ordc] * TM, 0, TM)
           * v_w).astype(jnp.int32)

    hb = hid // NH
    grid_spec = pltpu.PrefetchScalarGridSpec(
        num_scalar_prefetch=5,
        grid=(w_items, NH + 1),
        in_specs=[
            pl.BlockSpec((t_num, d), lambda w, h, *s: (0, 0)),
            pl.BlockSpec((1, d, hb),
                         lambda w, h, eid, rs, nv, tk, p:
                         (eid[w], 0, jnp.minimum(h, NH - 1))),
            pl.BlockSpec((1, d, hb),
                         lambda w, h, eid, rs, nv, tk, p:
                         (eid[w], 0, jnp.minimum(h, NH - 1))),
            pl.BlockSpec((1, hid, d),
                         lambda w, h, eid, rs, nv, tk, p: (eid[w], 0, 0)),
        ],
        out_specs=pl.BlockSpec((t_num, d), lambda w, h, *s: (0, 0)),
        scratch_shapes=[
            pltpu.VMEM((TM, d), jnp.float32),
            pltpu.VMEM((NH, TM, hb), jnp.bfloat16),
            pltpu.VMEM((TM, d), jnp.float32),
        ],
    )
    out = pl.pallas_call(
        functools.partial(_moe_body, nh=NH, hb=hb),
        grid_spec=grid_spec,
        out_shape=jax.ShapeDtypeStruct((t_num, d), jnp.float32),
        compiler_params=pltpu.CompilerParams(
            dimension_semantics=("arbitrary", "arbitrary"),
            vmem_limit_bytes=100 * 1024 * 1024,
        ),
    )(eidw, rsw, nvw, tok_s, p_s, xf, w1, w3, w2)

    return out.reshape(b, s, d)


# speedup vs baseline: 2.3004x; 1.0416x over previous
import functools

import jax
import jax.numpy as jnp
from jax.experimental import pallas as pl
from jax.experimental.pallas import tpu as pltpu

K = 2
TM = 512
NH = 2


def _moe_body(eid_ref, rs_ref, nv_ref, tok_ref, p_ref,
              xf_ref, w1_ref, w3_ref, w2_ref, out_ref,
              xs_ref, acc_ref, *, nh):
    w = pl.program_id(0)
    h = pl.program_id(1)

    @pl.when(jnp.logical_and(w == 0, h == 0))
    def _():
        out_ref[...] = jnp.zeros_like(out_ref)

    nv = nv_ref[w]
    rs = rs_ref[w]

    @pl.when(nv > 0)
    def _():
        @pl.when(h == 0)
        def _():
            def gather_row(i, carry):
                t = tok_ref[rs + i]
                xs_ref[pl.ds(i, 1), :] = xf_ref[pl.ds(t, 1), :]
                return carry
            jax.lax.fori_loop(0, nv, gather_row, 0)

        xb = xs_ref[...].astype(jnp.bfloat16)
        w1b = w1_ref[0].astype(jnp.bfloat16)
        w3b = w3_ref[0].astype(jnp.bfloat16)
        w2b = w2_ref[0].astype(jnp.bfloat16)
        g = jnp.dot(xb, w1b, preferred_element_type=jnp.float32)
        u = jnp.dot(xb, w3b, preferred_element_type=jnp.float32)
        hh = (g * jax.nn.sigmoid(g) * u).astype(jnp.bfloat16)
        part = jnp.dot(hh, w2b, preferred_element_type=jnp.float32)

        @pl.when(h == 0)
        def _():
            acc_ref[...] = part

        @pl.when(h != 0)
        def _():
            acc_ref[...] += part

        @pl.when(h == nh - 1)
        def _():
            def scatter_row(i, carry):
                r = rs + i
                t = tok_ref[r]
                out_ref[pl.ds(t, 1), :] += p_ref[r] * acc_ref[pl.ds(i, 1), :]
                return carry
            jax.lax.fori_loop(0, nv, scatter_row, 0)


def kernel(x, gate_w, w1, w3, w2):
    b, s, d = x.shape
    e_num, _, hid = w1.shape
    t_num = b * s
    r_num = t_num * K
    xf = x.reshape(t_num, d)

    scores = xf @ gate_w.T
    expert_weights, expert_indices = jax.lax.top_k(scores, K)
    expert_weights = jax.nn.softmax(expert_weights, axis=-1)

    ef = expert_indices.reshape(-1).astype(jnp.int32)
    onehot = (ef[:, None] == jnp.arange(e_num, dtype=jnp.int32)[None, :])
    cum = jnp.cumsum(onehot.astype(jnp.int32), axis=0)
    counts = cum[-1].astype(jnp.int32)
    starts = (jnp.cumsum(counts) - counts).astype(jnp.int32)
    rank = jnp.take_along_axis(cum, ef[:, None], axis=1)[:, 0] - 1
    slot = (starts[ef] + rank).astype(jnp.int32)
    tokf = jnp.arange(r_num, dtype=jnp.int32) // K
    tok_s = jnp.zeros((r_num,), jnp.int32).at[slot].set(tokf)
    p_s = jnp.zeros((r_num,), jnp.float32).at[slot].set(
        expert_weights.reshape(-1))

    maxj = r_num // TM
    w_items = e_num + maxj - 1
    nch = (counts + TM - 1) // TM
    e_c = jnp.repeat(jnp.arange(e_num, dtype=jnp.int32), maxj)
    j_c = jnp.tile(jnp.arange(maxj, dtype=jnp.int32), e_num)
    validc = j_c < nch[e_c]
    ordc = jnp.argsort(jnp.logical_not(validc).astype(jnp.int32))[:w_items]
    v_w = validc[ordc]
    eidw = jnp.where(v_w, e_c[ordc], e_num - 1).astype(jnp.int32)
    rsw = jnp.where(v_w, starts[e_c[ordc]] + j_c[ordc] * TM, 0).astype(jnp.int32)
    nvw = (jnp.clip(counts[e_c[ordc]] - j_c[ordc] * TM, 0, TM)
           * v_w).astype(jnp.int32)

    hb = hid // NH
    grid_spec = pltpu.PrefetchScalarGridSpec(
        num_scalar_prefetch=5,
        grid=(w_items, NH),
        in_specs=[
            pl.BlockSpec((t_num, d), lambda w, h, *s: (0, 0)),
            pl.BlockSpec((1, d, hb),
                         lambda w, h, eid, rs, nv, tk, p: (eid[w], 0, h)),
            pl.BlockSpec((1, d, hb),
                         lambda w, h, eid, rs, nv, tk, p: (eid[w], 0, h)),
            pl.BlockSpec((1, hb, d),
                         lambda w, h, eid, rs, nv, tk, p: (eid[w], h, 0)),
        ],
        out_specs=pl.BlockSpec((t_num, d), lambda w, h, *s: (0, 0)),
        scratch_shapes=[
            pltpu.VMEM((TM, d), jnp.float32),
            pltpu.VMEM((TM, d), jnp.float32),
        ],
    )
    out = pl.pallas_call(
        functools.partial(_moe_body, nh=NH),
        grid_spec=grid_spec,
        out_shape=jax.ShapeDtypeStruct((t_num, d), jnp.float32),
        compiler_params=pltpu.CompilerParams(
            dimension_semantics=("arbitrary", "arbitrary"),
            vmem_limit_bytes=100 * 1024 * 1024,
        ),
    )(eidw, rsw, nvw, tok_s, p_s, xf, w1, w3, w2)

    return out.reshape(b, s, d)
